# Initial kernel scaffold; baseline (speedup 1.0000x reference)
import jax
import jax.numpy as jnp
from jax.experimental import pallas as pl


def kernel(z, batch_index, W, b):
    raise NotImplementedError("write your pallas kernel here")


# speedup vs baseline: 7.7767x; 7.7767x over previous
import jax
import jax.numpy as jnp
from jax.experimental import pallas as pl
from jax.experimental.pallas import tpu as pltpu

_G = 512
_BLK = 2000


def _body(seg_ref, z_ref, w_ref, out_ref, acc_ref, cnt_ref, m_ref, d_ref):
    i = pl.program_id(0)
    nb = pl.num_programs(0)

    @pl.when(i == 0)
    def _():
        acc_ref[...] = jnp.zeros_like(acc_ref)
        cnt_ref[...] = jnp.zeros_like(cnt_ref)
        m_ref[...] = jnp.full_like(m_ref, -1e30)
        d_ref[...] = jnp.zeros_like(d_ref)

    z = z_ref[...]
    w = w_ref[...]
    s = jax.lax.dot_general(z, w, (((1,), (1,)), ((), ())),
                            preferred_element_type=jnp.float32)
    lm = jnp.max(s, keepdims=True)
    m_old = m_ref[...]
    m_new = jnp.maximum(m_old, lm)
    scale = jnp.exp(m_old - m_new)
    e = jnp.exp(s - m_new)

    seg = seg_ref[0]
    onehot = (jax.lax.broadcasted_iota(jnp.int32, (_G, _BLK), 0)
              == seg).astype(jnp.float32)
    acc_ref[...] = acc_ref[...] * scale + jax.lax.dot_general(
        onehot, e * z, (((1,), (0,)), ((), ())),
        preferred_element_type=jnp.float32)
    cnt_ref[...] += jnp.sum(onehot, axis=1, keepdims=True)
    d_ref[...] = d_ref[...] * scale + jnp.sum(e, keepdims=True)
    m_ref[...] = m_new

    @pl.when(i == nb - 1)
    def _():
        denom = d_ref[...] * jnp.maximum(cnt_ref[...], 1.0)
        out_ref[...] = acc_ref[...] / denom


@jax.jit
def _run(z, seg3, w):
    n, d = z.shape
    nb = n // _BLK
    return pl.pallas_call(
        _body,
        grid=(nb,),
        in_specs=[
            pl.BlockSpec((1, 1, _BLK), lambda i: (i, 0, 0)),
            pl.BlockSpec((_BLK, d), lambda i: (i, 0)),
            pl.BlockSpec((1, d), lambda i: (0, 0)),
        ],
        out_specs=pl.BlockSpec((_G, d), lambda i: (0, 0)),
        out_shape=jax.ShapeDtypeStruct((_G, d), jnp.float32),
        scratch_shapes=[
            pltpu.VMEM((_G, d), jnp.float32),
            pltpu.VMEM((_G, 1), jnp.float32),
            pltpu.VMEM((1, 1), jnp.float32),
            pltpu.VMEM((1, 1), jnp.float32),
        ],
    )(seg3, z, w)


def kernel(z, batch_index, W, b):
    n, _ = z.shape
    seg3 = batch_index.astype(jnp.int32).reshape(n // _BLK, 1, _BLK)
    return _run(z, seg3, W)


# speedup vs baseline: 8.0610x; 1.0366x over previous
import jax
import jax.numpy as jnp
from jax.experimental import pallas as pl
from jax.experimental.pallas import tpu as pltpu

_G = 512
_BLK = 2000
_SPAN = 128


def _body(seg_ref, z_ref, w_ref, out_ref, acc_ref, cnt_ref, m_ref, d_ref):
    i = pl.program_id(0)
    nb = pl.num_programs(0)

    @pl.when(i == 0)
    def _():
        acc_ref[...] = jnp.zeros_like(acc_ref)
        cnt_ref[...] = jnp.zeros_like(cnt_ref)
        m_ref[...] = jnp.full_like(m_ref, -1e30)
        d_ref[...] = jnp.zeros_like(d_ref)

    z = z_ref[...]
    w = w_ref[...]
    s = jax.lax.dot_general(z, w, (((1,), (1,)), ((), ())),
                            preferred_element_type=jnp.float32)
    lm = jnp.max(s, keepdims=True)
    m_old = m_ref[...]
    m_new = jnp.maximum(m_old, lm)
    scale = jnp.exp(m_old - m_new)
    e = jnp.exp(s - m_new)

    seg = seg_ref[0]
    ez = e * z
    acc_ref[...] = acc_ref[...] * scale
    d_ref[...] = d_ref[...] * scale + jnp.sum(e, keepdims=True)
    m_ref[...] = m_new

    smin = jnp.min(seg)
    smax = jnp.max(seg)
    s0 = jnp.minimum((smin // 8) * 8, _G - _SPAN)
    fast = (smax - s0) < _SPAN

    @pl.when(fast)
    def _():
        onehot = (jax.lax.broadcasted_iota(jnp.int32, (_SPAN, _BLK), 0)
                  == (seg - s0)).astype(jnp.float32)
        acc_ref[pl.ds(s0, _SPAN), :] += jax.lax.dot_general(
            onehot, ez, (((1,), (0,)), ((), ())),
            preferred_element_type=jnp.float32)
        cnt_ref[pl.ds(s0, _SPAN), :] += jnp.sum(onehot, axis=1, keepdims=True)

    @pl.when(jnp.logical_not(fast))
    def _():
        onehot = (jax.lax.broadcasted_iota(jnp.int32, (_G, _BLK), 0)
                  == seg).astype(jnp.float32)
        acc_ref[...] += jax.lax.dot_general(
            onehot, ez, (((1,), (0,)), ((), ())),
            preferred_element_type=jnp.float32)
        cnt_ref[...] += jnp.sum(onehot, axis=1, keepdims=True)

    @pl.when(i == nb - 1)
    def _():
        denom = d_ref[...] * jnp.maximum(cnt_ref[...], 1.0)
        out_ref[...] = acc_ref[...] / denom


@jax.jit
def _run(z, seg3, w):
    n, d = z.shape
    nb = n // _BLK
    return pl.pallas_call(
        _body,
        grid=(nb,),
        in_specs=[
            pl.BlockSpec((1, 1, _BLK), lambda i: (i, 0, 0)),
            pl.BlockSpec((_BLK, d), lambda i: (i, 0)),
            pl.BlockSpec((1, d), lambda i: (0, 0)),
        ],
        out_specs=pl.BlockSpec((_G, d), lambda i: (0, 0)),
        out_shape=jax.ShapeDtypeStruct((_G, d), jnp.float32),
        scratch_shapes=[
            pltpu.VMEM((_G, d), jnp.float32),
            pltpu.VMEM((_G, 1), jnp.float32),
            pltpu.VMEM((1, 1), jnp.float32),
            pltpu.VMEM((1, 1), jnp.float32),
        ],
    )(seg3, z, w)


def kernel(z, batch_index, W, b):
    n, _ = z.shape
    seg3 = batch_index.astype(jnp.int32).reshape(n // _BLK, 1, _BLK)
    return _run(z, seg3, W)


# speedup vs baseline: 8.4330x; 1.0461x over previous
import jax
import jax.numpy as jnp
from jax.experimental import pallas as pl
from jax.experimental.pallas import tpu as pltpu

_G = 512
_BLK = 2000
_SPAN = 64


def _body(seg_ref, z_ref, w_ref, out_ref, acc_ref, cnt_ref, m_ref, d_ref):
    i = pl.program_id(0)
    nb = pl.num_programs(0)

    @pl.when(i == 0)
    def _():
        acc_ref[...] = jnp.zeros_like(acc_ref)
        cnt_ref[...] = jnp.zeros_like(cnt_ref)
        m_ref[...] = jnp.full_like(m_ref, -1e30)
        d_ref[...] = jnp.zeros_like(d_ref)

    z = z_ref[...]
    w = w_ref[...]
    s = jax.lax.dot_general(z, w, (((1,), (1,)), ((), ())),
                            preferred_element_type=jnp.float32)
    lm = jnp.max(s, keepdims=True)
    m_old = m_ref[...]
    m_new = jnp.maximum(m_old, lm)
    scale = jnp.exp(m_old - m_new)
    e = jnp.exp(s - m_new)

    seg = seg_ref[0]
    ez = e * z
    acc_ref[...] = acc_ref[...] * scale
    d_ref[...] = d_ref[...] * scale + jnp.sum(e, keepdims=True)
    m_ref[...] = m_new

    smin = jnp.min(seg)
    smax = jnp.max(seg)
    s0 = jnp.minimum((smin // 8) * 8, _G - _SPAN)
    fast = (smax - s0) < _SPAN

    @pl.when(fast)
    def _():
        onehot = (jax.lax.broadcasted_iota(jnp.int32, (_SPAN, _BLK), 0)
                  == (seg - s0)).astype(jnp.bfloat16)
        acc_ref[pl.ds(s0, _SPAN), :] += jax.lax.dot_general(
            onehot, ez.astype(jnp.bfloat16), (((1,), (0,)), ((), ())),
            preferred_element_type=jnp.float32)
        cnt_ref[pl.ds(s0, _SPAN), :] += jnp.sum(
            onehot.astype(jnp.float32), axis=1, keepdims=True)

    @pl.when(jnp.logical_not(fast))
    def _():
        onehot = (jax.lax.broadcasted_iota(jnp.int32, (_G, _BLK), 0)
                  == seg).astype(jnp.float32)
        acc_ref[...] += jax.lax.dot_general(
            onehot, ez, (((1,), (0,)), ((), ())),
            preferred_element_type=jnp.float32)
        cnt_ref[...] += jnp.sum(onehot, axis=1, keepdims=True)

    @pl.when(i == nb - 1)
    def _():
        denom = d_ref[...] * jnp.maximum(cnt_ref[...], 1.0)
        out_ref[...] = acc_ref[...] / denom


@jax.jit
def _run(z, seg3, w):
    n, d = z.shape
    nb = n // _BLK
    return pl.pallas_call(
        _body,
        grid=(nb,),
        in_specs=[
            pl.BlockSpec((1, 1, _BLK), lambda i: (i, 0, 0)),
            pl.BlockSpec((_BLK, d), lambda i: (i, 0)),
            pl.BlockSpec((1, d), lambda i: (0, 0)),
        ],
        out_specs=pl.BlockSpec((_G, d), lambda i: (0, 0)),
        out_shape=jax.ShapeDtypeStruct((_G, d), jnp.float32),
        scratch_shapes=[
            pltpu.VMEM((_G, d), jnp.float32),
            pltpu.VMEM((_G, 1), jnp.float32),
            pltpu.VMEM((1, 1), jnp.float32),
            pltpu.VMEM((1, 1), jnp.float32),
        ],
    )(seg3, z, w)


def kernel(z, batch_index, W, b):
    n, _ = z.shape
    seg3 = batch_index.astype(jnp.int32).reshape(n // _BLK, 1, _BLK)
    return _run(z, seg3, W)


# speedup vs baseline: 9.9089x; 1.1750x over previous
import jax
import jax.numpy as jnp
from jax.experimental import pallas as pl
from jax.experimental.pallas import tpu as pltpu

_G = 512
_BLK = 2000
_SPAN = 64


def _body(seg_ref, z_ref, w_ref, out_ref, acc_ref, cnt_ref, m_ref, d_ref):
    i = pl.program_id(0)
    nb = pl.num_programs(0)

    @pl.when(i == 0)
    def _():
        acc_ref[...] = jnp.zeros_like(acc_ref)
        cnt_ref[...] = jnp.zeros_like(cnt_ref)
        m_ref[...] = jnp.full_like(m_ref, -1e30)
        d_ref[...] = jnp.zeros_like(d_ref)

    z = z_ref[...]
    w = w_ref[...]
    s = jax.lax.dot_general(w, z, (((1,), (1,)), ((), ())),
                            preferred_element_type=jnp.float32)
    lm = jnp.max(s, axis=1, keepdims=True)
    m_old = m_ref[...]
    m_new = jnp.maximum(m_old, lm)
    scale = jnp.exp(m_old - m_new)
    e = jnp.exp(s - m_new)

    seg = seg_ref[0]

    @pl.when(lm[0, 0] > m_old[0, 0])
    def _():
        acc_ref[...] = acc_ref[...] * scale

    d_ref[...] = d_ref[...] * scale + jnp.sum(e, keepdims=True)
    m_ref[...] = m_new

    smin = jnp.min(seg)
    smax = jnp.max(seg)
    s0 = jnp.minimum((smin // 8) * 8, _G - _SPAN)
    fast = (smax - s0) < _SPAN
    zb = z.astype(jnp.bfloat16)
    eb = e.astype(jnp.bfloat16)

    @pl.when(fast)
    def _():
        hit = (jax.lax.broadcasted_iota(jnp.int32, (_SPAN, _BLK), 0)
               == (seg - s0))
        wih = hit.astype(jnp.bfloat16) * eb
        acc_ref[pl.ds(s0, _SPAN), :] += jax.lax.dot_general(
            wih, zb, (((1,), (0,)), ((), ())),
            preferred_element_type=jnp.float32)
        cnt_ref[pl.ds(s0, _SPAN), :] += jnp.sum(
            hit.astype(jnp.float32), axis=1, keepdims=True)

    @pl.when(jnp.logical_not(fast))
    def _():
        hit = (jax.lax.broadcasted_iota(jnp.int32, (_G, _BLK), 0)
               == seg)
        wih = hit.astype(jnp.bfloat16) * eb
        acc_ref[...] += jax.lax.dot_general(
            wih, zb, (((1,), (0,)), ((), ())),
            preferred_element_type=jnp.float32)
        cnt_ref[...] += jnp.sum(hit.astype(jnp.float32), axis=1,
                                keepdims=True)

    @pl.when(i == nb - 1)
    def _():
        denom = d_ref[...] * jnp.maximum(cnt_ref[...], 1.0)
        out_ref[...] = acc_ref[...] / denom


@jax.jit
def _run(z, seg3, w):
    n, d = z.shape
    nb = n // _BLK
    return pl.pallas_call(
        _body,
        grid=(nb,),
        in_specs=[
            pl.BlockSpec((1, 1, _BLK), lambda i: (i, 0, 0)),
            pl.BlockSpec((_BLK, d), lambda i: (i, 0)),
            pl.BlockSpec((1, d), lambda i: (0, 0)),
        ],
        out_specs=pl.BlockSpec((_G, d), lambda i: (0, 0)),
        out_shape=jax.ShapeDtypeStruct((_G, d), jnp.float32),
        scratch_shapes=[
            pltpu.VMEM((_G, d), jnp.float32),
            pltpu.VMEM((_G, 1), jnp.float32),
            pltpu.VMEM((1, 1), jnp.float32),
            pltpu.VMEM((1, 1), jnp.float32),
        ],
    )(seg3, z, w)


def kernel(z, batch_index, W, b):
    n, _ = z.shape
    seg3 = batch_index.astype(jnp.int32).reshape(n // _BLK, 1, _BLK)
    return _run(z, seg3, W)


# speedup vs baseline: 14.9452x; 1.5083x over previous
import jax
import jax.numpy as jnp
from jax.experimental import pallas as pl
from jax.experimental.pallas import tpu as pltpu

_G = 512
_BLK = 4000
_SPAN = 64


def _body(seg_ref, z_ref, w_ref, out_ref, acc_ref, cnt_ref, m_ref, d_ref):
    i = pl.program_id(0)
    nb = pl.num_programs(0)

    @pl.when(i == 0)
    def _():
        acc_ref[...] = jnp.zeros_like(acc_ref)
        cnt_ref[...] = jnp.zeros_like(cnt_ref)
        m_ref[...] = jnp.full_like(m_ref, -1e30)
        d_ref[...] = jnp.zeros_like(d_ref)

    z = z_ref[...]
    w = w_ref[...]
    s = jax.lax.dot_general(w, z, (((1,), (1,)), ((), ())),
                            preferred_element_type=jnp.float32)
    lm = jnp.max(s, axis=1, keepdims=True)
    m_old = m_ref[...]
    m_new = jnp.maximum(m_old, lm)
    scale = jnp.exp(m_old - m_new)
    e = jnp.exp(s - m_new)

    seg = seg_ref[0]

    @pl.when(lm[0, 0] > m_old[0, 0])
    def _():
        acc_ref[...] = acc_ref[...] * scale

    d_ref[...] = d_ref[...] * scale + jnp.sum(e, keepdims=True)
    m_ref[...] = m_new

    smin = jnp.min(seg)
    smax = jnp.max(seg)
    s0 = jnp.minimum((smin // 8) * 8, _G - _SPAN)
    fast = (smax - s0) < _SPAN
    zb = z.astype(jnp.bfloat16)
    eb = e.astype(jnp.bfloat16)

    @pl.when(fast)
    def _():
        hit = (jax.lax.broadcasted_iota(jnp.int32, (_SPAN, _BLK), 0)
               == (seg - s0))
        wih = hit.astype(jnp.bfloat16) * eb
        acc_ref[pl.ds(s0, _SPAN), :] += jax.lax.dot_general(
            wih, zb, (((1,), (0,)), ((), ())),
            preferred_element_type=jnp.float32)
        cnt_ref[pl.ds(s0, _SPAN), :] += jnp.sum(
            hit.astype(jnp.float32), axis=1, keepdims=True)

    @pl.when(jnp.logical_not(fast))
    def _():
        hit = (jax.lax.broadcasted_iota(jnp.int32, (_G, _BLK), 0)
               == seg)
        wih = hit.astype(jnp.bfloat16) * eb
        acc_ref[...] += jax.lax.dot_general(
            wih, zb, (((1,), (0,)), ((), ())),
            preferred_element_type=jnp.float32)
        cnt_ref[...] += jnp.sum(hit.astype(jnp.float32), axis=1,
                                keepdims=True)

    @pl.when(i == nb - 1)
    def _():
        denom = d_ref[...] * jnp.maximum(cnt_ref[...], 1.0)
        out_ref[...] = acc_ref[...] / denom


@jax.jit
def _run(z, seg3, w):
    n, d = z.shape
    nb = n // _BLK
    return pl.pallas_call(
        _body,
        grid=(nb,),
        in_specs=[
            pl.BlockSpec((1, 1, _BLK), lambda i: (i, 0, 0)),
            pl.BlockSpec((_BLK, d), lambda i: (i, 0)),
            pl.BlockSpec((1, d), lambda i: (0, 0)),
        ],
        out_specs=pl.BlockSpec((_G, d), lambda i: (0, 0)),
        out_shape=jax.ShapeDtypeStruct((_G, d), jnp.float32),
        scratch_shapes=[
            pltpu.VMEM((_G, d), jnp.float32),
            pltpu.VMEM((_G, 1), jnp.float32),
            pltpu.VMEM((1, 1), jnp.float32),
            pltpu.VMEM((1, 1), jnp.float32),
        ],
    )(seg3, z, w)


def kernel(z, batch_index, W, b):
    n, _ = z.shape
    seg3 = batch_index.astype(jnp.int32).reshape(n // _BLK, 1, _BLK)
    return _run(z, seg3, W)


# speedup vs baseline: 16.6442x; 1.1137x over previous
import jax
import jax.numpy as jnp
from jax.experimental import pallas as pl
from jax.experimental.pallas import tpu as pltpu

_G = 512
_BLK = 5000
_SPAN = 64


def _body(seg_ref, z_ref, w_ref, out_ref, acc_ref, cnt_ref, m_ref, d_ref):
    i = pl.program_id(0)
    nb = pl.num_programs(0)

    @pl.when(i == 0)
    def _():
        acc_ref[...] = jnp.zeros_like(acc_ref)
        cnt_ref[...] = jnp.zeros_like(cnt_ref)
        m_ref[...] = jnp.full_like(m_ref, -1e30)
        d_ref[...] = jnp.zeros_like(d_ref)

    z = z_ref[...]
    w = w_ref[...]
    s = jax.lax.dot_general(w, z, (((1,), (1,)), ((), ())),
                            preferred_element_type=jnp.float32)
    lm = jnp.max(s, axis=1, keepdims=True)
    m_old = m_ref[...]
    m_new = jnp.maximum(m_old, lm)
    scale = jnp.exp(m_old - m_new)
    e = jnp.exp(s - m_new)

    seg = seg_ref[0]

    @pl.when(lm[0, 0] > m_old[0, 0])
    def _():
        acc_ref[...] = acc_ref[...] * scale

    d_ref[...] = d_ref[...] * scale + jnp.sum(e, keepdims=True)
    m_ref[...] = m_new

    smin = jnp.min(seg)
    smax = jnp.max(seg)
    s0 = jnp.minimum((smin // 8) * 8, _G - _SPAN)
    fast = (smax - s0) < _SPAN
    zb = z.astype(jnp.bfloat16)
    eb = e.astype(jnp.bfloat16)

    @pl.when(fast)
    def _():
        hit = (jax.lax.broadcasted_iota(jnp.int32, (_SPAN, _BLK), 0)
               == (seg - s0))
        wih = hit.astype(jnp.bfloat16) * eb
        acc_ref[pl.ds(s0, _SPAN), :] += jax.lax.dot_general(
            wih, zb, (((1,), (0,)), ((), ())),
            preferred_element_type=jnp.float32)
        cnt_ref[pl.ds(s0, _SPAN), :] += jnp.sum(
            hit.astype(jnp.float32), axis=1, keepdims=True)

    @pl.when(jnp.logical_not(fast))
    def _():
        hit = (jax.lax.broadcasted_iota(jnp.int32, (_G, _BLK), 0)
               == seg)
        wih = hit.astype(jnp.bfloat16) * eb
        acc_ref[...] += jax.lax.dot_general(
            wih, zb, (((1,), (0,)), ((), ())),
            preferred_element_type=jnp.float32)
        cnt_ref[...] += jnp.sum(hit.astype(jnp.float32), axis=1,
                                keepdims=True)

    @pl.when(i == nb - 1)
    def _():
        denom = d_ref[...] * jnp.maximum(cnt_ref[...], 1.0)
        out_ref[...] = acc_ref[...] / denom


@jax.jit
def _run(z, seg3, w):
    n, d = z.shape
    nb = n // _BLK
    return pl.pallas_call(
        _body,
        grid=(nb,),
        in_specs=[
            pl.BlockSpec((1, 1, _BLK), lambda i: (i, 0, 0)),
            pl.BlockSpec((_BLK, d), lambda i: (i, 0)),
            pl.BlockSpec((1, d), lambda i: (0, 0)),
        ],
        out_specs=pl.BlockSpec((_G, d), lambda i: (0, 0)),
        out_shape=jax.ShapeDtypeStruct((_G, d), jnp.float32),
        scratch_shapes=[
            pltpu.VMEM((_G, d), jnp.float32),
            pltpu.VMEM((_G, 1), jnp.float32),
            pltpu.VMEM((1, 1), jnp.float32),
            pltpu.VMEM((1, 1), jnp.float32),
        ],
    )(seg3, z, w)


def kernel(z, batch_index, W, b):
    n, _ = z.shape
    seg3 = batch_index.astype(jnp.int32).reshape(n // _BLK, 1, _BLK)
    return _run(z, seg3, W)


# speedup vs baseline: 19.3908x; 1.1650x over previous
import jax
import jax.numpy as jnp
from jax.experimental import pallas as pl
from jax.experimental.pallas import tpu as pltpu

_G = 512
_BLK = 10000
_SPAN = 128


def _body(seg_ref, z_ref, w_ref, out_ref, acc_ref, cnt_ref, m_ref, d_ref):
    i = pl.program_id(0)
    nb = pl.num_programs(0)

    @pl.when(i == 0)
    def _():
        acc_ref[...] = jnp.zeros_like(acc_ref)
        cnt_ref[...] = jnp.zeros_like(cnt_ref)
        m_ref[...] = jnp.full_like(m_ref, -1e30)
        d_ref[...] = jnp.zeros_like(d_ref)

    z = z_ref[...]
    w = w_ref[...]
    s = jax.lax.dot_general(w, z, (((1,), (1,)), ((), ())),
                            preferred_element_type=jnp.float32)
    lm = jnp.max(s, axis=1, keepdims=True)
    m_old = m_ref[...]
    m_new = jnp.maximum(m_old, lm)
    scale = jnp.exp(m_old - m_new)
    e = jnp.exp(s - m_new)

    seg = seg_ref[0]

    @pl.when(lm[0, 0] > m_old[0, 0])
    def _():
        acc_ref[...] = acc_ref[...] * scale

    d_ref[...] = d_ref[...] * scale + jnp.sum(e, keepdims=True)
    m_ref[...] = m_new

    smin = jnp.min(seg)
    smax = jnp.max(seg)
    s0 = jnp.minimum((smin // 8) * 8, _G - _SPAN)
    fast = (smax - s0) < _SPAN
    zb = z.astype(jnp.bfloat16)
    eb = e.astype(jnp.bfloat16)

    @pl.when(fast)
    def _():
        hit = (jax.lax.broadcasted_iota(jnp.int32, (_SPAN, _BLK), 0)
               == (seg - s0))
        wih = hit.astype(jnp.bfloat16) * eb
        acc_ref[pl.ds(s0, _SPAN), :] += jax.lax.dot_general(
            wih, zb, (((1,), (0,)), ((), ())),
            preferred_element_type=jnp.float32)
        cnt_ref[pl.ds(s0, _SPAN), :] += jnp.sum(
            hit.astype(jnp.float32), axis=1, keepdims=True)

    @pl.when(jnp.logical_not(fast))
    def _():
        hit = (jax.lax.broadcasted_iota(jnp.int32, (_G, _BLK), 0)
               == seg)
        wih = hit.astype(jnp.bfloat16) * eb
        acc_ref[...] += jax.lax.dot_general(
            wih, zb, (((1,), (0,)), ((), ())),
            preferred_element_type=jnp.float32)
        cnt_ref[...] += jnp.sum(hit.astype(jnp.float32), axis=1,
                                keepdims=True)

    @pl.when(i == nb - 1)
    def _():
        denom = d_ref[...] * jnp.maximum(cnt_ref[...], 1.0)
        out_ref[...] = acc_ref[...] / denom


@jax.jit
def _run(z, seg3, w):
    n, d = z.shape
    nb = n // _BLK
    return pl.pallas_call(
        _body,
        grid=(nb,),
        in_specs=[
            pl.BlockSpec((1, 1, _BLK), lambda i: (i, 0, 0)),
            pl.BlockSpec((_BLK, d), lambda i: (i, 0)),
            pl.BlockSpec((1, d), lambda i: (0, 0)),
        ],
        out_specs=pl.BlockSpec((_G, d), lambda i: (0, 0)),
        out_shape=jax.ShapeDtypeStruct((_G, d), jnp.float32),
        scratch_shapes=[
            pltpu.VMEM((_G, d), jnp.float32),
            pltpu.VMEM((_G, 1), jnp.float32),
            pltpu.VMEM((1, 1), jnp.float32),
            pltpu.VMEM((1, 1), jnp.float32),
        ],
    )(seg3, z, w)


def kernel(z, batch_index, W, b):
    n, _ = z.shape
    seg3 = batch_index.astype(jnp.int32).reshape(n // _BLK, 1, _BLK)
    return _run(z, seg3, W)
